# baseline (speedup 1.0000x reference)
import functools

import jax
import jax.numpy as jnp
from jax import lax
from jax.experimental import pallas as pl
from jax.experimental.pallas import tpu as pltpu
from jax.experimental.pallas import tpu_sc as plsc

NC, NS = 2, 16
NW = NC * NS
CHUNK = 128
ALIGN = NW * CHUNK * 8


def _cdiv(a, b):
    return (a + b - 1) // b


def _mesh():
    return plsc.VectorSubcoreMesh(core_axis_name="c", subcore_axis_name="s",
                                  num_cores=NC, num_subcores=NS)


def _sc_gather(u, idx2d, rows_out):
    gw = rows_out // (NW * CHUNK)
    w = u.shape[1]

    nb = 4

    def body(u_hbm, idx_hbm, out_hbm, idxs, *bufs):
        rows = bufs[:nb]
        gs = bufs[nb:2 * nb]
        ws = bufs[2 * nb:3 * nb]
        wid = lax.axis_index("s") * NC + lax.axis_index("c")
        pltpu.sync_copy(idx_hbm.at[pl.ds(wid * gw, gw)], idxs)

        for b in range(nb):
            pltpu.async_copy(u_hbm.at[idxs.at[b]], rows[b], gs[b])

        @pl.loop(0, gw // nb)
        def it(c):
            for b in range(nb):
                cc = nb * c + b
                pltpu.make_async_copy(u_hbm.at[idxs.at[cc]], rows[b], gs[b]).wait()
                pltpu.async_copy(
                    rows[b], out_hbm.at[pl.ds((wid * gw + cc) * CHUNK, CHUNK)], ws[b])
            for b in range(nb):
                cc = nb * c + b + nb

                @pl.when(cc < gw)
                def _():
                    pltpu.make_async_copy(rows[b], out_hbm.at[pl.ds(0, CHUNK)], ws[b]).wait()
                    pltpu.async_copy(u_hbm.at[idxs.at[cc]], rows[b], gs[b])

        for b in range(nb):
            pltpu.make_async_copy(rows[b], out_hbm.at[pl.ds(0, CHUNK)], ws[b]).wait()

    f = pl.kernel(
        body,
        out_type=jax.ShapeDtypeStruct((rows_out, w), u.dtype),
        mesh=_mesh(),
        scratch_types=(
            [pltpu.VMEM((gw, CHUNK), jnp.int32)]
            + [pltpu.VMEM((CHUNK, w), u.dtype)] * nb
            + [pltpu.SemaphoreType.DMA] * (2 * nb)
        ),
    )
    return f(u, idx2d)


def _sc_scatter(msg, dst2d, ep, n_out, width):
    cw = ep // (NW * CHUNK)
    nsh = n_out + 8
    r_per = _cdiv(_cdiv(n_out, NS), 8) * 8
    r_last = n_out - (NS - 1) * r_per
    z_last = nsh - (NS - 1) * r_per
    counts_mode = msg is None

    nb = 2

    def body(*refs):
        if counts_mode:
            ones_hbm, dst_hbm, out_hbm, idxs = refs[:4]
        else:
            msg_hbm, dst_hbm, out_hbm, idxs = refs[:4]
        rows = refs[4:4 + nb]
        zbuf = refs[4 + nb]
        shared = refs[5 + nb]
        ls = refs[6 + nb:6 + 2 * nb]
        as_ = refs[6 + 2 * nb:6 + 3 * nb]
        cid = lax.axis_index("c")
        sid = lax.axis_index("s")
        wid = sid * NC + cid

        for r in range(8):
            for j in range(width // 16):
                zbuf[r, pl.ds(j * 16, 16)] = jnp.zeros((16,), jnp.float32)
        if counts_mode:
            pltpu.sync_copy(ones_hbm, rows[0])

        nz = jnp.where(sid == NS - 1, z_last // 8, r_per // 8)

        @pl.loop(0, nz)
        def zero(k):
            pltpu.sync_copy(zbuf, shared.at[pl.ds(sid * r_per + k * 8, 8)])

        plsc.subcore_barrier()

        pltpu.sync_copy(dst_hbm.at[pl.ds(wid * cw, cw)], idxs)

        if counts_mode:
            @pl.loop(0, cw)
            def chunk(c):
                pltpu.sync_copy(rows[0], shared.at[idxs.at[c]], add=True)
        else:
            for b in range(nb):
                pltpu.async_copy(
                    msg_hbm.at[pl.ds((wid * cw + b) * CHUNK, CHUNK)], rows[b], ls[b])

            @pl.loop(0, cw // nb)
            def chunk(c):
                for b in range(nb):
                    cc = nb * c + b
                    pltpu.make_async_copy(
                        msg_hbm.at[pl.ds(0, CHUNK)], rows[b], ls[b]).wait()
                    pltpu.async_copy(rows[b], shared.at[idxs.at[cc]], as_[b], add=True)
                for b in range(nb):
                    cc = nb * c + b + nb

                    @pl.when(cc < cw)
                    def _():
                        pltpu.make_async_copy(rows[b], shared.at[pl.ds(0, CHUNK)], as_[b]).wait()
                        pltpu.async_copy(
                            msg_hbm.at[pl.ds((wid * cw + cc) * CHUNK, CHUNK)], rows[b], ls[b])

            for b in range(nb):
                pltpu.make_async_copy(rows[b], shared.at[pl.ds(0, CHUNK)], as_[b]).wait()

        plsc.subcore_barrier()

        @pl.when(sid < NS - 1)
        def _():
            pltpu.sync_copy(shared.at[pl.ds(sid * r_per, r_per)],
                            out_hbm.at[cid].at[pl.ds(sid * r_per, r_per)])

        @pl.when(sid == NS - 1)
        def _():
            pltpu.sync_copy(shared.at[pl.ds((NS - 1) * r_per, r_last)],
                            out_hbm.at[cid].at[pl.ds((NS - 1) * r_per, r_last)])

    f = pl.kernel(
        body,
        out_type=jax.ShapeDtypeStruct((NC, n_out, width), jnp.float32),
        mesh=_mesh(),
        scratch_types=(
            [pltpu.VMEM((cw, CHUNK), jnp.int32)]
            + [pltpu.VMEM((CHUNK, width), jnp.float32)] * nb
            + [pltpu.VMEM((8, width), jnp.float32),
               pltpu.VMEM_SHARED((nsh, width), jnp.float32)]
            + [pltpu.SemaphoreType.DMA] * (2 * nb)
        ),
    )
    if counts_mode:
        return f(jnp.ones((CHUNK, width), jnp.float32), dst2d)
    return f(msg, dst2d)


def _mm_body(x_ref, w_ref, o_ref):
    o_ref[...] = jnp.dot(x_ref[...], w_ref[...],
                         preferred_element_type=jnp.float32).astype(o_ref.dtype)


def _tc_mm(x, w, out_dtype=jnp.float32):
    return pl.pallas_call(
        _mm_body,
        out_shape=jax.ShapeDtypeStruct((x.shape[0], w.shape[1]), out_dtype),
    )(x, w)


def _tail_body(asrc, adst, ea, w1e, b1, w2, b2, w3, b3, o):
    h = asrc[...].astype(jnp.float32) + adst[...].astype(jnp.float32) + b1[...]
    h = h + jnp.dot(ea[...], w1e[...], preferred_element_type=jnp.float32)
    h = jnp.maximum(h, 0.0)
    h = jnp.maximum(jnp.dot(h, w2[...], preferred_element_type=jnp.float32) + b2[...], 0.0)
    o[...] = jnp.dot(h, w3[...], preferred_element_type=jnp.float32) + b3[...]


def _tail(a, eap, w1e, b1, w2, b2, w3, b3, ep, bt=2048):
    nb = ep // bt
    wspec = lambda s: pl.BlockSpec(s, lambda i: (0, 0))
    return pl.pallas_call(
        _tail_body,
        grid=(nb,),
        in_specs=[
            pl.BlockSpec((bt, 128), lambda i: (i, 0)),
            pl.BlockSpec((bt, 128), lambda i: (i + nb, 0)),
            pl.BlockSpec((bt, 16), lambda i: (i, 0)),
            wspec((16, 128)), wspec((1, 128)),
            wspec((128, 128)), wspec((1, 128)),
            wspec((128, 128)), wspec((1, 128)),
        ],
        out_specs=pl.BlockSpec((bt, 128), lambda i: (i, 0)),
        out_shape=jax.ShapeDtypeStruct((ep, 128), jnp.float32),
    )(a, a, eap, w1e, b1, w2, b2, w3, b3)


def _inv_cnt_body(cnt, o):
    c = cnt[0] + cnt[1]
    o[...] = 1.0 / jnp.maximum(c, 1.0)


def _inv_cnt(cnt):
    n = cnt.shape[1]
    return pl.pallas_call(
        _inv_cnt_body,
        out_shape=jax.ShapeDtypeStruct((n, 128), jnp.float32),
    )(cnt)


def _combine_mid_body(res, p, inv, vprev, sb, w, vout, uout):
    mean = (p[0] + p[1]) * inv[...]
    v = mean + vprev[...] if res else mean
    v = v * sb[0:1, :] + sb[1:2, :]
    vout[...] = v
    uout[...] = jnp.dot(v, w[...],
                        preferred_element_type=jnp.float32).astype(uout.dtype)


def _combine_mid(part, cnt, vprev, sb, w, res):
    n = part.shape[1]
    return pl.pallas_call(
        functools.partial(_combine_mid_body, res),
        out_shape=(jax.ShapeDtypeStruct((n, 128), jnp.float32),
                   jax.ShapeDtypeStruct((n, 128), jnp.float32)),
    )(part, cnt, vprev, sb, w)


def _combine_final_body(p, inv, fout):
    fout[...] = (p[0] + p[1]) * inv[...]


def _combine_final(part, cnt):
    n = part.shape[1]
    return pl.pallas_call(
        _combine_final_body,
        out_shape=jax.ShapeDtypeStruct((n, 128), jnp.float32),
    )(part, cnt)


def kernel(x, edge_index, edge_attr, params):
    n, d = x.shape
    e = edge_index.shape[1]
    ep = _cdiv(e, ALIGN) * ALIGN
    pad = ep - e

    src = edge_index[0].astype(jnp.int32)
    dst = edge_index[1].astype(jnp.int32)
    order = jnp.argsort(dst)
    src = src[order]
    dst = dst[order]
    edge_attr = edge_attr[order]
    srcp = jnp.concatenate([src, jnp.zeros((pad,), jnp.int32)])
    dstp = jnp.concatenate([dst, jnp.full((pad,), n, jnp.int32)])
    idx2d = jnp.concatenate([srcp, jnp.minimum(dstp, n - 1)]).reshape(-1, CHUNK)
    dst2d = dstp.reshape(-1, CHUNK)
    eap = jnp.concatenate(
        [edge_attr, jnp.zeros((pad, edge_attr.shape[1]), edge_attr.dtype)])

    p = params
    inv_bn = 1.0 / jnp.sqrt(jnp.float32(1.0 + 1e-5))
    ones_r = jnp.ones((1, 128), jnp.float32)
    zeros_r = jnp.zeros((1, 128), jnp.float32)
    sb = [
        jnp.concatenate([(p["bn1_g"] * inv_bn).reshape(1, -1), p["bn1_b"].reshape(1, -1)]),
        jnp.concatenate([(p["bn2_g"] * inv_bn).reshape(1, -1), p["bn2_b"].reshape(1, -1)]),
        jnp.concatenate([ones_r, zeros_r]),
    ]
    mlps = [p["mlp0"], p["mlp1"], p["mlp2"], p["mlp3"]]
    w1v = [m["W1"][:d] for m in mlps]
    w1e = [m["W1"][d:] for m in mlps]

    cnt = _inv_cnt(_sc_scatter(None, dst2d, ep, n, 128))
    u = _tc_mm(x, w1v[0])
    v = x
    f = None
    for k in range(4):
        m = mlps[k]
        a = _sc_gather(u, idx2d, 2 * ep)
        msg = _tail(a, eap, w1e[k], m["b1"].reshape(1, -1), m["W2"],
                    m["b2"].reshape(1, -1), m["W3"], m["b3"].reshape(1, -1), ep)
        part = _sc_scatter(msg, dst2d, ep, n, 128)
        if k < 3:
            v, u = _combine_mid(part, cnt, v, sb[k], w1v[k + 1], res=(k > 0))
        else:
            f = _combine_final(part, cnt)
    return f


# speedup vs baseline: 1.3258x; 1.3258x over previous
import functools

import jax
import jax.numpy as jnp
from jax import lax
from jax.experimental import pallas as pl
from jax.experimental.pallas import tpu as pltpu
from jax.experimental.pallas import tpu_sc as plsc

NC, NS = 2, 16
NW = NC * NS
CHUNK = 128
ALIGN = NW * CHUNK * 8


def _cdiv(a, b):
    return (a + b - 1) // b


def _mesh():
    return plsc.VectorSubcoreMesh(core_axis_name="c", subcore_axis_name="s",
                                  num_cores=NC, num_subcores=NS)


def _sc_gather(u, idx2d, rows_out):
    gw = rows_out // (NW * CHUNK)
    w = u.shape[1]

    nb = 4

    def body(u_hbm, idx_hbm, out_hbm, idxs, *bufs):
        rows = bufs[:nb]
        gs = bufs[nb:2 * nb]
        ws = bufs[2 * nb:3 * nb]
        wid = lax.axis_index("s") * NC + lax.axis_index("c")
        pltpu.sync_copy(idx_hbm.at[pl.ds(wid * gw, gw)], idxs)

        for b in range(nb):
            pltpu.async_copy(u_hbm.at[idxs.at[b]], rows[b], gs[b])

        @pl.loop(0, gw // nb)
        def it(c):
            for b in range(nb):
                cc = nb * c + b
                pltpu.make_async_copy(u_hbm.at[idxs.at[cc]], rows[b], gs[b]).wait()
                pltpu.async_copy(
                    rows[b], out_hbm.at[pl.ds((wid * gw + cc) * CHUNK, CHUNK)], ws[b])
            for b in range(nb):
                cc = nb * c + b + nb

                @pl.when(cc < gw)
                def _():
                    pltpu.make_async_copy(rows[b], out_hbm.at[pl.ds(0, CHUNK)], ws[b]).wait()
                    pltpu.async_copy(u_hbm.at[idxs.at[cc]], rows[b], gs[b])

        for b in range(nb):
            pltpu.make_async_copy(rows[b], out_hbm.at[pl.ds(0, CHUNK)], ws[b]).wait()

    f = pl.kernel(
        body,
        out_type=jax.ShapeDtypeStruct((rows_out, w), u.dtype),
        mesh=_mesh(),
        scratch_types=(
            [pltpu.VMEM((gw, CHUNK), jnp.int32)]
            + [pltpu.VMEM((CHUNK, w), u.dtype)] * nb
            + [pltpu.SemaphoreType.DMA] * (2 * nb)
        ),
    )
    return f(u, idx2d)


def _sc_scatter(msg, dst2d, ep, n_out, width):
    cw = ep // (NW * CHUNK)
    nsh = n_out + 8
    r_per = _cdiv(_cdiv(n_out, NS), 8) * 8
    r_last = n_out - (NS - 1) * r_per
    z_last = nsh - (NS - 1) * r_per
    counts_mode = msg is None

    nb = 2

    def body(*refs):
        if counts_mode:
            ones_hbm, dst_hbm, out_hbm, idxs = refs[:4]
            nin = 4
        else:
            msga_hbm, msgb_hbm, dst_hbm, out_hbm, idxs = refs[:5]
            nin = 5
        rows = refs[nin:nin + nb]
        zbuf = refs[nin + nb]
        shared = refs[nin + 1 + nb]
        ls = refs[nin + 2 + nb:nin + 2 + 2 * nb]
        as_ = refs[nin + 2 + 2 * nb:nin + 2 + 3 * nb]
        cid = lax.axis_index("c")
        sid = lax.axis_index("s")
        wid = sid * NC + cid

        for r in range(8):
            for j in range(width // 16):
                zbuf[r, pl.ds(j * 16, 16)] = jnp.zeros((16,), jnp.float32)
        if counts_mode:
            pltpu.sync_copy(ones_hbm, rows[0])

        nz = jnp.where(sid == NS - 1, z_last // 8, r_per // 8)

        @pl.loop(0, nz)
        def zero(k):
            pltpu.sync_copy(zbuf, shared.at[pl.ds(sid * r_per + k * 8, 8)])

        plsc.subcore_barrier()

        pltpu.sync_copy(dst_hbm.at[pl.ds(wid * cw, cw)], idxs)

        if counts_mode:
            @pl.loop(0, cw)
            def chunk(c):
                pltpu.sync_copy(rows[0], shared.at[idxs.at[c]], add=True)
        else:
            hw = NW // 2
            loc = jnp.where(wid < hw, wid, wid - hw) * cw

            def half(msg_hbm):
                for b in range(nb):
                    pltpu.async_copy(
                        msg_hbm.at[pl.ds((loc + b) * CHUNK, CHUNK)], rows[b], ls[b])

                @pl.loop(0, cw // nb)
                def chunk(c):
                    for b in range(nb):
                        cc = nb * c + b
                        pltpu.make_async_copy(
                            msg_hbm.at[pl.ds(0, CHUNK)], rows[b], ls[b]).wait()
                        pltpu.async_copy(rows[b], shared.at[idxs.at[cc]], as_[b], add=True)
                    for b in range(nb):
                        cc = nb * c + b + nb

                        @pl.when(cc < cw)
                        def _():
                            pltpu.make_async_copy(rows[b], shared.at[pl.ds(0, CHUNK)], as_[b]).wait()
                            pltpu.async_copy(
                                msg_hbm.at[pl.ds((loc + cc) * CHUNK, CHUNK)], rows[b], ls[b])

                for b in range(nb):
                    pltpu.make_async_copy(rows[b], shared.at[pl.ds(0, CHUNK)], as_[b]).wait()

            @pl.when(wid < hw)
            def _():
                half(msga_hbm)

            @pl.when(wid >= hw)
            def _():
                half(msgb_hbm)

        plsc.subcore_barrier()

        @pl.when(sid < NS - 1)
        def _():
            pltpu.sync_copy(shared.at[pl.ds(sid * r_per, r_per)],
                            out_hbm.at[cid].at[pl.ds(sid * r_per, r_per)])

        @pl.when(sid == NS - 1)
        def _():
            pltpu.sync_copy(shared.at[pl.ds((NS - 1) * r_per, r_last)],
                            out_hbm.at[cid].at[pl.ds((NS - 1) * r_per, r_last)])

    f = pl.kernel(
        body,
        out_type=jax.ShapeDtypeStruct((NC, n_out, width), jnp.float32),
        mesh=_mesh(),
        scratch_types=(
            [pltpu.VMEM((cw, CHUNK), jnp.int32)]
            + [pltpu.VMEM((CHUNK, width), jnp.float32)] * nb
            + [pltpu.VMEM((8, width), jnp.float32),
               pltpu.VMEM_SHARED((nsh, width), jnp.float32)]
            + [pltpu.SemaphoreType.DMA] * (2 * nb)
        ),
    )
    if counts_mode:
        return f(jnp.ones((CHUNK, width), jnp.float32), dst2d)
    return f(msg[0], msg[1], dst2d)


def _mm_body(x_ref, w_ref, o_ref):
    o_ref[...] = jnp.dot(x_ref[...], w_ref[...],
                         preferred_element_type=jnp.float32).astype(o_ref.dtype)


def _tc_mm(x, w, out_dtype=jnp.float32):
    return pl.pallas_call(
        _mm_body,
        out_shape=jax.ShapeDtypeStruct((x.shape[0], w.shape[1]), out_dtype),
    )(x, w)


def _tail_body(asrc, adst, ea, w1e, b1, w2, b2, w3, b3, o):
    h = asrc[...].astype(jnp.float32) + adst[...].astype(jnp.float32) + b1[...]
    h = h + jnp.dot(ea[...], w1e[...], preferred_element_type=jnp.float32)
    h = jnp.maximum(h, 0.0)
    h = jnp.maximum(jnp.dot(h, w2[...], preferred_element_type=jnp.float32) + b2[...], 0.0)
    o[...] = jnp.dot(h, w3[...], preferred_element_type=jnp.float32) + b3[...]


def _tail(a, eap, w1e, b1, w2, b2, w3, b3, ep, eoff=0, bt=2048):
    nb = ep // bt
    wspec = lambda s: pl.BlockSpec(s, lambda i: (0, 0))
    return pl.pallas_call(
        _tail_body,
        grid=(nb,),
        in_specs=[
            pl.BlockSpec((bt, 128), lambda i: (i, 0)),
            pl.BlockSpec((bt, 128), lambda i: (i + nb, 0)),
            pl.BlockSpec((bt, 16), lambda i: (i + eoff, 0)),
            wspec((16, 128)), wspec((1, 128)),
            wspec((128, 128)), wspec((1, 128)),
            wspec((128, 128)), wspec((1, 128)),
        ],
        out_specs=pl.BlockSpec((bt, 128), lambda i: (i, 0)),
        out_shape=jax.ShapeDtypeStruct((ep, 128), jnp.float32),
    )(a, a, eap, w1e, b1, w2, b2, w3, b3)


def _inv_cnt_body(cnt, o):
    c = cnt[0] + cnt[1]
    o[...] = 1.0 / jnp.maximum(c, 1.0)


def _inv_cnt(cnt):
    n = cnt.shape[1]
    return pl.pallas_call(
        _inv_cnt_body,
        out_shape=jax.ShapeDtypeStruct((n, 128), jnp.float32),
    )(cnt)


def _combine_mid_body(res, p, inv, vprev, sb, w, vout, uout):
    mean = (p[0] + p[1]) * inv[...]
    v = mean + vprev[...] if res else mean
    v = v * sb[0:1, :] + sb[1:2, :]
    vout[...] = v
    uout[...] = jnp.dot(v, w[...],
                        preferred_element_type=jnp.float32).astype(uout.dtype)


def _combine_mid(part, cnt, vprev, sb, w, res):
    n = part.shape[1]
    return pl.pallas_call(
        functools.partial(_combine_mid_body, res),
        out_shape=(jax.ShapeDtypeStruct((n, 128), jnp.float32),
                   jax.ShapeDtypeStruct((n, 128), jnp.float32)),
    )(part, cnt, vprev, sb, w)


def _combine_final_body(p, inv, fout):
    fout[...] = (p[0] + p[1]) * inv[...]


def _combine_final(part, cnt):
    n = part.shape[1]
    return pl.pallas_call(
        _combine_final_body,
        out_shape=jax.ShapeDtypeStruct((n, 128), jnp.float32),
    )(part, cnt)


def kernel(x, edge_index, edge_attr, params):
    n, d = x.shape
    e = edge_index.shape[1]
    ep = _cdiv(e, ALIGN) * ALIGN
    pad = ep - e

    src = edge_index[0].astype(jnp.int32)
    dst = edge_index[1].astype(jnp.int32)
    srcp = jnp.concatenate([src, jnp.zeros((pad,), jnp.int32)])
    dstp = jnp.concatenate([dst, jnp.full((pad,), n, jnp.int32)])
    dstg = jnp.minimum(dstp, n - 1)
    eph = ep // 2
    idx2d_h = [
        jnp.concatenate([srcp[:eph], dstg[:eph]]).reshape(-1, CHUNK),
        jnp.concatenate([srcp[eph:], dstg[eph:]]).reshape(-1, CHUNK),
    ]
    dst2d = dstp.reshape(-1, CHUNK)
    eap = jnp.concatenate(
        [edge_attr, jnp.zeros((pad, edge_attr.shape[1]), edge_attr.dtype)])

    p = params
    inv_bn = 1.0 / jnp.sqrt(jnp.float32(1.0 + 1e-5))
    ones_r = jnp.ones((1, 128), jnp.float32)
    zeros_r = jnp.zeros((1, 128), jnp.float32)
    sb = [
        jnp.concatenate([(p["bn1_g"] * inv_bn).reshape(1, -1), p["bn1_b"].reshape(1, -1)]),
        jnp.concatenate([(p["bn2_g"] * inv_bn).reshape(1, -1), p["bn2_b"].reshape(1, -1)]),
        jnp.concatenate([ones_r, zeros_r]),
    ]
    mlps = [p["mlp0"], p["mlp1"], p["mlp2"], p["mlp3"]]
    w1v = [m["W1"][:d] for m in mlps]
    w1e = [m["W1"][d:] for m in mlps]

    cnt = _inv_cnt(_sc_scatter(None, dst2d, ep, n, 128))
    u = _tc_mm(x, w1v[0])
    v = x
    f = None
    eph = ep // 2
    for k in range(4):
        m = mlps[k]
        msgs = []
        for h in range(2):
            a = _sc_gather(u, idx2d_h[h], ep)
            msgs.append(_tail(a, eap, w1e[k], m["b1"].reshape(1, -1), m["W2"],
                              m["b2"].reshape(1, -1), m["W3"],
                              m["b3"].reshape(1, -1), eph,
                              eoff=h * (eph // 2048)))
        part = _sc_scatter(tuple(msgs), dst2d, ep, n, 128)
        if k < 3:
            v, u = _combine_mid(part, cnt, v, sb[k], w1v[k + 1], res=(k > 0))
        else:
            f = _combine_final(part, cnt)
    return f
